# Initial kernel scaffold; baseline (speedup 1.0000x reference)
import jax
import jax.numpy as jnp
from jax.experimental import pallas as pl


def kernel(feat16, feat8, feat4, params):
    raise NotImplementedError("write your pallas kernel here")


# speedup vs baseline: 1.0696x; 1.0696x over previous
import jax
import jax.numpy as jnp
from jax.experimental import pallas as pl
from jax.experimental.pallas import tpu as pltpu

_HEADS = ('dep', 'dim', 'hm', 'reg', 'rot', 'wh')
_TILE = 32
_HW = 128


def _body(x_ref, w_ref, b_ref, w2_ref, b2_ref,
          o_dep, o_dim, o_reg, o_rot, o_wh, o_hm):
    r = pl.program_id(1)
    base = r * _TILE
    n = _TILE * _HW
    accs = [jnp.zeros((n, 64), jnp.float32) for _ in range(6)]
    for t in range(9):
        dy, dx = divmod(t, 3)
        xs = x_ref[0, pl.ds(base + dy, _TILE), dx:dx + _HW, :].reshape(n, 64)
        for j in range(6):
            accs[j] = accs[j] + jnp.dot(xs, w_ref[j, t],
                                        preferred_element_type=jnp.float32)
    outs = [o_dep, o_dim, None, o_reg, o_rot, o_wh]
    for j in range(6):
        y = jnp.maximum(accs[j] + b_ref[j], 0.0)
        if j == 2:
            z = jax.nn.sigmoid(jnp.dot(y, w2_ref[...],
                                       preferred_element_type=jnp.float32)
                               + b2_ref[...])
            o_hm[0] = z.T
        else:
            outs[j][0] = y.T


def kernel(feat16, feat8, feat4, params):
    del feat16, feat8
    b, c, h, w = feat4.shape
    s = 1.0 / jnp.sqrt(1.0 + 1e-5)

    w_taps = []
    biases = []
    for name in _HEADS:
        p = params[name]
        scale = p['gamma'] * s
        w1 = p['w1'] * scale[:, None, None, None]
        w_taps.append(jnp.transpose(w1, (2, 3, 1, 0)).reshape(9, 64, 64))
        biases.append(p['b1'] * scale + p['beta'])
    w_all = jnp.stack(w_taps).astype(jnp.bfloat16)
    b_all = jnp.stack(biases)[:, None, :]

    w2 = params['hm']['w2'][:, :, 0, 0]
    w2p = jnp.zeros((64, 8), jnp.float32).at[:, :3].set(w2.T)
    b2p = jnp.zeros((1, 8), jnp.float32).at[0, :3].set(params['hm']['b2'])

    xp = jnp.pad(jnp.transpose(feat4, (0, 2, 3, 1)),
                 ((0, 0), (1, 1), (1, 1), (0, 0))).astype(jnp.bfloat16)

    n_tiles = h // _TILE
    hw = h * w
    out_sd = jax.ShapeDtypeStruct((b, 64, hw), jnp.float32)
    grid = (b, n_tiles)

    outs = pl.pallas_call(
        _body,
        grid=grid,
        in_specs=[
            pl.BlockSpec((1, h + 2, w + 2, 64), lambda bi, r: (bi, 0, 0, 0)),
            pl.BlockSpec((6, 9, 64, 64), lambda bi, r: (0, 0, 0, 0)),
            pl.BlockSpec((6, 1, 64), lambda bi, r: (0, 0, 0)),
            pl.BlockSpec((64, 8), lambda bi, r: (0, 0)),
            pl.BlockSpec((1, 8), lambda bi, r: (0, 0)),
        ],
        out_specs=[
            pl.BlockSpec((1, 64, _TILE * w), lambda bi, r: (bi, 0, r)),
            pl.BlockSpec((1, 64, _TILE * w), lambda bi, r: (bi, 0, r)),
            pl.BlockSpec((1, 64, _TILE * w), lambda bi, r: (bi, 0, r)),
            pl.BlockSpec((1, 64, _TILE * w), lambda bi, r: (bi, 0, r)),
            pl.BlockSpec((1, 64, _TILE * w), lambda bi, r: (bi, 0, r)),
            pl.BlockSpec((1, 8, _TILE * w), lambda bi, r: (bi, 0, r)),
        ],
        out_shape=[out_sd, out_sd, out_sd, out_sd, out_sd,
                   jax.ShapeDtypeStruct((b, 8, hw), jnp.float32)],
        compiler_params=pltpu.CompilerParams(
            vmem_limit_bytes=100 * 1024 * 1024),
    )(xp, w_all, b_all, w2p, b2p)

    dep, dim, reg, rot, wh, hm8 = outs
    shape4 = (b, 64, h, w)
    hm = hm8.reshape(b, 8, h, w)[:, :3]
    return (dep.reshape(shape4), dim.reshape(shape4), hm,
            reg.reshape(shape4), rot.reshape(shape4), wh.reshape(shape4))
